# baseline (device time: 35806 ns/iter reference)
import numpy as np
import jax
import jax.numpy as jnp
from jax import lax
from jax.experimental import pallas as pl
from jax.experimental.pallas import tpu as pltpu

ND = 32
B, S, D = 2, 128, 512
DH = 64
R = B * S
CH = R // ND


def kernel(x, Wq, Wk, Wv, Wo):
    d_local = Wq.shape[1]
    HL = d_local // DH

    inv = 1.0 / (10000.0 ** (np.arange(0, DH, 2) / DH))
    ang = np.arange(S)[:, None] * inv[None, :]
    cos = np.tile(np.repeat(np.cos(ang), 2, axis=1), (1, HL)).astype(np.float32)
    sin = np.tile(np.repeat(np.sin(ang), 2, axis=1), (1, HL)).astype(np.float32)

    def body(x_ref, wq_ref, wk_ref, wv_ref, wo_ref, cos_ref, sin_ref,
             out_ref, p_ref, acc_ref, comm_ref, send1, recv1, send2, recv2):
        me = lax.axis_index("i")

        bar = pltpu.get_barrier_semaphore()
        for d in range(1, ND):
            pl.semaphore_signal(
                bar, inc=1, device_id=((me + d) % ND,),
                device_id_type=pl.DeviceIdType.MESH,
            )

        cos_t = cos_ref[...]
        sin_t = sin_ref[...]
        cosq_t = cos_t * 0.125
        sinq_t = sin_t * 0.125

        col = lax.broadcasted_iota(jnp.int32, cos_t.shape, 1)
        even = (col % 2) == 0

        def rope(t, c, sn):
            tr = jnp.where(even, -jnp.roll(t, -1, axis=1),
                           jnp.roll(t, 1, axis=1))
            return t * c + tr * sn

        def attn_batch(b):
            xb = x_ref[b]
            q = jnp.dot(xb, wq_ref[...], preferred_element_type=jnp.float32)
            k = jnp.dot(xb, wk_ref[...], preferred_element_type=jnp.float32)
            v = jnp.dot(xb, wv_ref[...], preferred_element_type=jnp.float32)
            q = rope(q, cosq_t, sinq_t)
            k = rope(k, cos_t, sin_t)
            ctxs = []
            for h in range(HL):
                qh = q[:, h * DH:(h + 1) * DH]
                kh = k[:, h * DH:(h + 1) * DH]
                s = jnp.dot(qh, kh.T, preferred_element_type=jnp.float32)
                s = s - jnp.max(s, axis=-1, keepdims=True)
                w = jnp.exp(s)
                w = w / jnp.sum(w, axis=-1, keepdims=True)
                ctxs.append(jnp.dot(w, v[:, h * DH:(h + 1) * DH],
                                    preferred_element_type=jnp.float32))
            ctx = jnp.concatenate(ctxs, axis=1)
            p_ref[pl.ds(b * S, S), :] = jnp.dot(
                ctx, wo_ref[...], preferred_element_type=jnp.float32)


        rdma1 = []
        js = []
        for d in range(1, ND):
            j = (me + d) % ND
            js.append(j)
            rdma1.append(pltpu.make_async_remote_copy(
                src_ref=p_ref.at[pl.ds(j * CH, CH), :],
                dst_ref=comm_ref.at[d],
                send_sem=send1.at[d],
                recv_sem=recv1.at[d],
                device_id=(j,),
                device_id_type=pl.DeviceIdType.MESH,
            ))

        half_chunks = (B * S // 2) // CH

        attn_batch(0)
        pl.semaphore_wait(bar, ND - 1)
        for snd, j in zip(rdma1, js):
            @pl.when(j < half_chunks)
            def _():
                snd.start()

        attn_batch(1)
        for snd, j in zip(rdma1, js):
            @pl.when(j >= half_chunks)
            def _():
                snd.start()

        for d in range(1, ND):
            rdma1[d - 1].wait_recv()
        terms = [p_ref[pl.ds(me * CH, CH), :]]
        terms += [comm_ref[d] for d in range(1, ND)]
        while len(terms) > 1:
            nxt = [terms[i] + terms[i + 1] for i in range(0, len(terms) - 1, 2)]
            if len(terms) % 2:
                nxt.append(terms[-1])
            terms = nxt
        acc = terms[0]
        acc_ref[...] = acc
        my_b = me // (S // CH)
        my_r = (me % (S // CH)) * CH

        rdma2 = []
        for d in range(1, ND):
            j = (me + d) % ND
            snd = pltpu.make_async_remote_copy(
                src_ref=acc_ref,
                dst_ref=out_ref.at[my_b, pl.ds(my_r, CH), :],
                send_sem=send2.at[d],
                recv_sem=recv2.at[d],
                device_id=(j,),
                device_id_type=pl.DeviceIdType.MESH,
            )
            snd.start()
            rdma2.append(snd)
        out_ref[my_b, pl.ds(my_r, CH), :] = acc

        for d in range(1, ND):
            rdma2[d - 1].wait_recv()
        for r in rdma1:
            r.wait_send()
        for r in rdma2:
            r.wait_send()

    return pl.pallas_call(
        body,
        out_shape=jax.ShapeDtypeStruct((B, S, D), jnp.float32),
        in_specs=[pl.BlockSpec(memory_space=pltpu.VMEM)] * 7,
        out_specs=pl.BlockSpec(memory_space=pltpu.VMEM),
        scratch_shapes=[
            pltpu.VMEM((R, D), jnp.float32),
            pltpu.VMEM((CH, D), jnp.float32),
            pltpu.VMEM((ND, CH, D), jnp.float32),
            pltpu.SemaphoreType.DMA((ND,)),
            pltpu.SemaphoreType.DMA((ND,)),
            pltpu.SemaphoreType.DMA((ND,)),
            pltpu.SemaphoreType.DMA((ND,)),
        ],
        compiler_params=pltpu.CompilerParams(collective_id=0),
    )(x, Wq, Wk, Wv, Wo, jnp.asarray(cos), jnp.asarray(sin))
